# Initial kernel scaffold; baseline (speedup 1.0000x reference)
import jax
import jax.numpy as jnp
from jax.experimental import pallas as pl


def kernel(x, table):
    raise NotImplementedError("write your pallas kernel here")


# speedup vs baseline: 1.1108x; 1.1108x over previous
import functools

import jax
import jax.numpy as jnp
from jax import lax
from jax.experimental import pallas as pl
from jax.experimental.pallas import tpu as pltpu
from jax.experimental.pallas import tpu_sc as plsc

_D = 32
_CHUNK = 128
_K = 10


@functools.lru_cache(maxsize=None)
def _make_gather(num_rows: int, b_total: int):
    info = plsc.get_sparse_core_info()
    nw = info.num_cores * info.num_subcores
    bpw = b_total // nw
    nch = bpw // _CHUNK
    ng = nch // _K
    grp = _K * _CHUNK
    assert b_total == nw * ng * grp and ng % 2 == 0

    mesh = plsc.VectorSubcoreMesh(core_axis_name="c", subcore_axis_name="s")

    @functools.partial(
        pl.kernel,
        mesh=mesh,
        out_type=jax.ShapeDtypeStruct((b_total, _D), jnp.float32),
        compiler_params=pltpu.CompilerParams(use_tc_tiling_on_sc=False),
        scratch_types=[
            pltpu.VMEM((nch, _CHUNK), jnp.int32),
            pltpu.VMEM((2, grp, _D), jnp.float32),
            pltpu.SemaphoreType.DMA,
            pltpu.SemaphoreType.DMA,
        ],
    )
    def body(idx_hbm, table_hbm, out_hbm, idx_v, rows_v, gsem, wsem):
        c = lax.axis_index("c")
        s = lax.axis_index("s")
        wid = s * info.num_cores + c
        base = wid * bpw

        pltpu.sync_copy(idx_hbm.at[wid], idx_v)

        for b in range(_K):
            pltpu.async_copy(
                table_hbm.at[idx_v.at[b]],
                rows_v.at[0, pl.ds(b * _CHUNK, _CHUNK)],
                gsem,
            )

        def group_step(go, carry):
            for p in range(2):
                g = go * 2 + p
                pltpu.make_async_copy(
                    table_hbm.at[pl.ds(0, grp)], rows_v.at[p], gsem
                ).wait()

                @pl.when(g >= 1)
                def _():
                    pltpu.make_async_copy(
                        rows_v.at[1 - p], out_hbm.at[pl.ds(0, grp)], wsem
                    ).wait()

                @pl.when(g + 1 < ng)
                def _():
                    for b in range(_K):
                        pltpu.async_copy(
                            table_hbm.at[idx_v.at[(g + 1) * _K + b]],
                            rows_v.at[1 - p, pl.ds(b * _CHUNK, _CHUNK)],
                            gsem,
                        )

                pltpu.async_copy(
                    rows_v.at[p], out_hbm.at[pl.ds(base + g * grp, grp)], wsem
                )
            return carry

        lax.fori_loop(0, ng // 2, group_step, 0)

        pltpu.make_async_copy(
            rows_v.at[1], out_hbm.at[pl.ds(0, grp)], wsem
        ).wait()

    return body


def kernel(x, table):
    b_total = x.shape[0] * x.shape[1]
    idx = x.reshape(-1).astype(jnp.int32)
    info = plsc.get_sparse_core_info()
    nw = info.num_cores * info.num_subcores
    nch = b_total // (nw * _CHUNK)
    idx3 = idx.reshape(nw, nch, _CHUNK)
    out = _make_gather(table.shape[0], b_total)(idx3, table)
    return out.reshape(x.shape + (_D,))


# speedup vs baseline: 1.1131x; 1.0021x over previous
import functools

import jax
import jax.numpy as jnp
from jax import lax
from jax.experimental import pallas as pl
from jax.experimental.pallas import tpu as pltpu
from jax.experimental.pallas import tpu_sc as plsc

_D = 32
_CHUNK = 128
_K = 5
_NB = 4


@functools.lru_cache(maxsize=None)
def _make_gather(num_rows: int, b_total: int):
    info = plsc.get_sparse_core_info()
    nw = info.num_cores * info.num_subcores
    bpw = b_total // nw
    nch = bpw // _CHUNK
    ng = nch // _K
    grp = _K * _CHUNK
    assert b_total == nw * ng * grp and ng % _NB == 0

    mesh = plsc.VectorSubcoreMesh(core_axis_name="c", subcore_axis_name="s")

    @functools.partial(
        pl.kernel,
        mesh=mesh,
        out_type=jax.ShapeDtypeStruct((b_total, _D), jnp.float32),
        compiler_params=pltpu.CompilerParams(use_tc_tiling_on_sc=False),
        scratch_types=[
            pltpu.VMEM((nch, _CHUNK), jnp.int32),
            pltpu.VMEM((_NB, grp, _D), jnp.float32),
            pltpu.SemaphoreType.DMA,
            pltpu.SemaphoreType.DMA,
        ],
    )
    def body(idx_hbm, table_hbm, out_hbm, idx_v, rows_v, gsem, wsem):
        c = lax.axis_index("c")
        s = lax.axis_index("s")
        wid = s * info.num_cores + c
        base = wid * bpw

        pltpu.sync_copy(idx_hbm.at[wid], idx_v)

        def start_gathers(g, p):
            for b in range(_K):
                pltpu.async_copy(
                    table_hbm.at[idx_v.at[g * _K + b]],
                    rows_v.at[p, pl.ds(b * _CHUNK, _CHUNK)],
                    gsem,
                )

        for q in range(_NB - 1):
            start_gathers(q, q)

        def group_step(go, carry):
            for p in range(_NB):
                g = go * _NB + p
                pltpu.make_async_copy(
                    table_hbm.at[pl.ds(0, grp)], rows_v.at[p], gsem
                ).wait()

                @pl.when(g >= 1)
                def _():
                    pltpu.make_async_copy(
                        rows_v.at[p], out_hbm.at[pl.ds(0, grp)], wsem
                    ).wait()

                @pl.when(g + _NB - 1 < ng)
                def _():
                    start_gathers(g + _NB - 1, (p + _NB - 1) % _NB)

                pltpu.async_copy(
                    rows_v.at[p], out_hbm.at[pl.ds(base + g * grp, grp)], wsem
                )
            return carry

        lax.fori_loop(0, ng // _NB, group_step, 0)

        pltpu.make_async_copy(
            rows_v.at[_NB - 1], out_hbm.at[pl.ds(0, grp)], wsem
        ).wait()

    return body


def kernel(x, table):
    b_total = x.shape[0] * x.shape[1]
    idx = x.reshape(-1).astype(jnp.int32)
    info = plsc.get_sparse_core_info()
    nw = info.num_cores * info.num_subcores
    nch = b_total // (nw * _CHUNK)
    idx3 = idx.reshape(nw, nch, _CHUNK)
    out = _make_gather(table.shape[0], b_total)(idx3, table)
    return out.reshape(x.shape + (_D,))


# speedup vs baseline: 1.8452x; 1.6577x over previous
import functools

import jax
import jax.numpy as jnp
from jax import lax
from jax.experimental import pallas as pl
from jax.experimental.pallas import tpu as pltpu
from jax.experimental.pallas import tpu_sc as plsc

_D = 32
_CHUNK = 128
_J = 50
_IT = 128


@functools.lru_cache(maxsize=None)
def _make_gather(num_rows: int, b_total: int):
    info = plsc.get_sparse_core_info()
    nw = info.num_cores * info.num_subcores
    itw = _IT // nw
    nblk = _J * itw
    assert b_total == nw * nblk * _CHUNK

    mesh = plsc.VectorSubcoreMesh(core_axis_name="c", subcore_axis_name="s")

    @functools.partial(
        pl.kernel,
        mesh=mesh,
        out_type=jax.ShapeDtypeStruct((b_total * _D,), jnp.float32),
        compiler_params=pltpu.CompilerParams(
            use_tc_tiling_on_sc=False, needs_layout_passes=False
        ),
        scratch_types=[
            pltpu.VMEM((nblk, _CHUNK), jnp.int32),
            pltpu.VMEM((2 * itw, _CHUNK, _D), jnp.float32),
            pltpu.VMEM((2 * itw * _CHUNK * _D,), jnp.float32),
            pltpu.SemaphoreType.DMA,
            pltpu.SemaphoreType.DMA,
        ],
    )
    def body(idx_hbm, table_hbm, out_hbm, idx_v, rows_v, jbuf, gsem, wsem):
        c = lax.axis_index("c")
        s = lax.axis_index("s")
        wid = s * info.num_cores + c

        pltpu.sync_copy(idx_hbm.at[wid], idx_v)

        lane_d = [lax.iota(jnp.int32, 16) + 16 * h for h in range(2)]
        svecs = [
            [(d // 8) * (itw * 1024) + k * 1024 + (d % 8) * _CHUNK
             for d in lane_d]
            for k in range(itw)
        ]

        def start_gather(n, slot):
            pltpu.async_copy(
                table_hbm.at[idx_v.at[n]], rows_v.at[slot], gsem
            )

        nring = 2 * itw
        for n in range(nring):
            start_gather(n, n)

        def jgroup(j2, carry):
            for p in range(2):
                jj = j2 * 2 + p

                @pl.when(jj >= 2)
                def _():
                    pltpu.make_async_copy(
                        jbuf.at[pl.ds(p * 16384, 16384)],
                        out_hbm.at[pl.ds(0, itw * _CHUNK * _D)],
                        wsem,
                    ).wait()

                for k in range(itw):
                    n = jj * itw + k
                    slot = p * itw + k

                    pltpu.make_async_copy(
                        table_hbm.at[pl.ds(0, _CHUNK)], rows_v.at[slot], gsem
                    ).wait()

                    def trans(t, carry2):
                        for u in range(8):
                            ii = t * 8 + u
                            for h in range(2):
                                v = rows_v[slot, ii, pl.ds(h * 16, 16)]
                                plsc.store_scatter(
                                    jbuf,
                                    [svecs[k][h] + (p * 16384 + ii)], v
                                )
                        return carry2

                    lax.fori_loop(0, _CHUNK // 8, trans, 0)

                    @pl.when(n + nring < nblk)
                    def _():
                        start_gather(n + nring, slot)

                for dt in range(4):
                    off = ((jj * 4 + dt) * _IT + itw * wid) * (8 * _CHUNK)
                    pltpu.async_copy(
                        jbuf.at[pl.ds(p * 16384 + dt * itw * 8 * _CHUNK,
                                      itw * 8 * _CHUNK)],
                        out_hbm.at[pl.ds(off, itw * 8 * _CHUNK)],
                        wsem,
                    )
            return carry

        lax.fori_loop(0, _J // 2, jgroup, 0)

        for p in range(2):
            pltpu.make_async_copy(
                jbuf.at[pl.ds(p * 16384, 16384)],
                out_hbm.at[pl.ds(0, itw * _CHUNK * _D)], wsem
            ).wait()

    return body


def kernel(x, table):
    b_total = x.shape[0] * x.shape[1]
    idx = x.T.astype(jnp.int32)
    info = plsc.get_sparse_core_info()
    nw = info.num_cores * info.num_subcores
    itw = _IT // nw
    idx3 = (idx.reshape(_J, nw, itw, _CHUNK)
            .transpose(1, 0, 2, 3)
            .reshape(nw, _J * itw, _CHUNK))
    flat = _make_gather(table.shape[0], b_total)(idx3, table)
    out5 = flat.reshape(_J, 4, _IT, 8, _CHUNK)
    return out5.transpose(2, 4, 0, 1, 3).reshape(x.shape + (_D,))
